# Initial kernel scaffold; baseline (speedup 1.0000x reference)
import jax
import jax.numpy as jnp
from jax.experimental import pallas as pl


def kernel(category_ids, weight):
    raise NotImplementedError("write your pallas kernel here")


# speedup vs baseline: 1.1087x; 1.1087x over previous
import functools

import jax
import jax.numpy as jnp
from jax import lax
from jax.experimental import pallas as pl
from jax.experimental.pallas import tpu as pltpu
from jax.experimental.pallas import tpu_sc as plsc

_NC = 2
_NS = 16
_NW = _NC * _NS
_CHUNK = 2560


def kernel(category_ids, weight):
    batch, hist = category_ids.shape
    num_idx = batch * hist
    dim = weight.shape[1]
    per_w = num_idx // _NW
    n_chunks = per_w // _CHUNK
    flat_idx = category_ids.reshape(num_idx)

    mesh = plsc.VectorSubcoreMesh(core_axis_name="c", subcore_axis_name="s")

    @functools.partial(
        pl.kernel, mesh=mesh,
        compiler_params=pltpu.CompilerParams(use_tc_tiling_on_sc=False),
        out_type=jax.ShapeDtypeStruct((num_idx, dim), weight.dtype),
        scratch_types=[
            pltpu.VMEM((_CHUNK,), jnp.int32),
            pltpu.VMEM((_CHUNK, dim), jnp.float32),
            pltpu.SemaphoreType.DMA,
        ],
    )
    def _gather(table_hbm, idx_hbm, out_hbm, idx_v, rows_v, sem):
        wid = lax.axis_index("s") * _NC + lax.axis_index("c")
        wbase = wid * per_w

        @pl.loop(0, n_chunks)
        def _(ci):
            base = wbase + ci * _CHUNK
            pltpu.sync_copy(idx_hbm.at[pl.ds(base, _CHUNK)], idx_v)
            pltpu.async_copy(table_hbm.at[idx_v], rows_v, sem).wait()
            pltpu.sync_copy(rows_v, out_hbm.at[pl.ds(base, _CHUNK)])

    flat_out = _gather(weight, flat_idx)
    return flat_out.reshape(batch, hist, dim)


# speedup vs baseline: 1.1130x; 1.0039x over previous
import functools

import jax
import jax.numpy as jnp
from jax import lax
from jax.experimental import pallas as pl
from jax.experimental.pallas import tpu as pltpu
from jax.experimental.pallas import tpu_sc as plsc

_NC = 2
_NS = 16
_NW = _NC * _NS
_CHUNK = 800
_NBUF = 4


def kernel(category_ids, weight):
    batch, hist = category_ids.shape
    num_idx = batch * hist
    dim = weight.shape[1]
    per_w = num_idx // _NW
    n_chunks = per_w // _CHUNK
    n_groups = n_chunks // _NBUF
    assert per_w % _CHUNK == 0 and n_chunks % _NBUF == 0 and n_groups >= 2
    flat_idx = category_ids.reshape(num_idx)

    mesh = plsc.VectorSubcoreMesh(core_axis_name="c", subcore_axis_name="s")

    scratch = (
        [pltpu.VMEM((_CHUNK,), jnp.int32) for _ in range(_NBUF)]
        + [pltpu.VMEM((_CHUNK, dim), jnp.float32) for _ in range(_NBUF)]
        + [pltpu.SemaphoreType.DMA for _ in range(3 * _NBUF)]
    )

    @functools.partial(
        pl.kernel, mesh=mesh,
        compiler_params=pltpu.CompilerParams(use_tc_tiling_on_sc=False),
        out_type=jax.ShapeDtypeStruct((num_idx, dim), weight.dtype),
        scratch_types=scratch,
    )
    def _gather(table_hbm, idx_hbm, out_hbm, *refs):
        idx_v = refs[:_NBUF]
        rows_v = refs[_NBUF:2 * _NBUF]
        sem_i = refs[2 * _NBUF:3 * _NBUF]
        sem_g = refs[3 * _NBUF:4 * _NBUF]
        sem_o = refs[4 * _NBUF:5 * _NBUF]

        wid = lax.axis_index("s") * _NC + lax.axis_index("c")
        wbase = wid * per_w

        def start_idx(ci, b):
            pltpu.make_async_copy(
                idx_hbm.at[pl.ds(wbase + ci * _CHUNK, _CHUNK)],
                idx_v[b], sem_i[b]).start()

        def wait_idx(b):
            pltpu.make_async_copy(
                idx_hbm.at[pl.ds(wbase, _CHUNK)], idx_v[b], sem_i[b]).wait()

        def start_gather(b):
            pltpu.make_async_copy(table_hbm.at[idx_v[b]], rows_v[b],
                                  sem_g[b]).start()

        def wait_gather(b):
            pltpu.make_async_copy(table_hbm.at[idx_v[b]], rows_v[b],
                                  sem_g[b]).wait()

        def start_out(ci, b):
            pltpu.make_async_copy(
                rows_v[b],
                out_hbm.at[pl.ds(wbase + ci * _CHUNK, _CHUNK)],
                sem_o[b]).start()

        def wait_out(b):
            pltpu.make_async_copy(
                rows_v[b], out_hbm.at[pl.ds(wbase, _CHUNK)],
                sem_o[b]).wait()

        for b in range(_NBUF):
            start_idx(b, b)

        for b in range(_NBUF):
            wait_idx(b)
            start_gather(b)
            if b >= 1:
                bp = b - 1
                wait_gather(bp)
                start_out(bp, bp)
                start_idx(bp + _NBUF, bp)

        @pl.loop(1, n_groups - 1)
        def _(g):
            for b in range(_NBUF):
                ci = g * _NBUF + b
                wait_idx(b)
                wait_out(b)
                start_gather(b)
                bp = (b - 1) % _NBUF
                wait_gather(bp)
                start_out(ci - 1, bp)
                start_idx(ci - 1 + _NBUF, bp)

        g_last = n_groups - 1
        for b in range(_NBUF):
            ci = g_last * _NBUF + b
            wait_idx(b)
            wait_out(b)
            start_gather(b)
            bp = (b - 1) % _NBUF
            wait_gather(bp)
            start_out(ci - 1, bp)
            if ci - 1 + _NBUF < n_chunks:
                start_idx(ci - 1 + _NBUF, bp)

        b_last = _NBUF - 1
        wait_gather(b_last)
        start_out(n_chunks - 1, b_last)
        for b in range(_NBUF):
            wait_out(b)

    flat_out = _gather(weight, flat_idx)
    return flat_out.reshape(batch, hist, dim)


# speedup vs baseline: 1.8076x; 1.6241x over previous
import functools

import jax
import jax.numpy as jnp
from jax import lax
from jax.experimental import pallas as pl
from jax.experimental.pallas import tpu as pltpu
from jax.experimental.pallas import tpu_sc as plsc

_NC = 2
_NS = 16
_NW = _NC * _NS
_CHUNK = 800
_NBUF = 4


def kernel(category_ids, weight):
    batch, hist = category_ids.shape
    num_idx = batch * hist
    dim = weight.shape[1]
    per_w = num_idx // _NW
    n_chunks = per_w // _CHUNK
    n_groups = n_chunks // _NBUF
    assert num_idx % _NW == 0 and per_w % _CHUNK == 0
    assert n_chunks % _NBUF == 0 and n_groups >= 2

    mesh = plsc.VectorSubcoreMesh(core_axis_name="c", subcore_axis_name="s")

    scratch = (
        [pltpu.VMEM((_CHUNK,), jnp.int32) for _ in range(_NBUF)]
        + [pltpu.VMEM((_CHUNK, dim), jnp.float32) for _ in range(_NBUF)]
        + [pltpu.SemaphoreType.DMA for _ in range(3 * _NBUF)]
    )

    @functools.partial(
        pl.kernel, mesh=mesh,
        compiler_params=pltpu.CompilerParams(use_tc_tiling_on_sc=False),
        out_type=jax.ShapeDtypeStruct((batch, hist, dim), weight.dtype),
        scratch_types=scratch,
    )
    def _gather(table_hbm, idx_hbm, out_hbm, *refs):
        idx_v = refs[:_NBUF]
        rows_v = refs[_NBUF:2 * _NBUF]
        sem_i = refs[2 * _NBUF:3 * _NBUF]
        sem_g = refs[3 * _NBUF:4 * _NBUF]
        sem_o = refs[4 * _NBUF:5 * _NBUF]

        wid = lax.axis_index("s") * _NC + lax.axis_index("c")
        wbase = wid * per_w

        def start_idx(ci, b):
            pltpu.make_async_copy(
                idx_hbm.at[pl.ds(wbase + ci * _CHUNK, _CHUNK)],
                idx_v[b], sem_i[b]).start()

        def wait_idx(b):
            pltpu.make_async_copy(
                idx_hbm.at[pl.ds(wbase, _CHUNK)], idx_v[b], sem_i[b]).wait()

        def start_gather(b):
            pltpu.make_async_copy(table_hbm.at[idx_v[b]], rows_v[b],
                                  sem_g[b]).start()

        def wait_gather(b):
            pltpu.make_async_copy(table_hbm.at[idx_v[b]], rows_v[b],
                                  sem_g[b]).wait()

        rows_per_chunk = _CHUNK // hist
        wbase_rows = wid * (per_w // hist)

        def start_out(ci, b):
            row0 = wbase_rows + ci * rows_per_chunk
            for r in range(rows_per_chunk):
                pltpu.make_async_copy(
                    rows_v[b].at[pl.ds(r * hist, hist)],
                    out_hbm.at[row0 + r],
                    sem_o[b]).start()

        def wait_out(b):
            for r in range(rows_per_chunk):
                pltpu.make_async_copy(
                    rows_v[b].at[pl.ds(r * hist, hist)],
                    out_hbm.at[wbase_rows + r],
                    sem_o[b]).wait()

        for b in range(_NBUF):
            start_idx(b, b)

        for b in range(_NBUF):
            wait_idx(b)
            start_gather(b)
            if b >= 1:
                bp = b - 1
                wait_gather(bp)
                start_out(bp, bp)
                start_idx(bp + _NBUF, bp)

        @pl.loop(1, n_groups - 1)
        def _(g):
            for b in range(_NBUF):
                ci = g * _NBUF + b
                wait_idx(b)
                wait_out(b)
                start_gather(b)
                bp = (b - 1) % _NBUF
                wait_gather(bp)
                start_out(ci - 1, bp)
                start_idx(ci - 1 + _NBUF, bp)

        g_last = n_groups - 1
        for b in range(_NBUF):
            ci = g_last * _NBUF + b
            wait_idx(b)
            wait_out(b)
            start_gather(b)
            bp = (b - 1) % _NBUF
            wait_gather(bp)
            start_out(ci - 1, bp)
            if ci - 1 + _NBUF < n_chunks:
                start_idx(ci - 1 + _NBUF, bp)

        b_last = _NBUF - 1
        wait_gather(b_last)
        start_out(n_chunks - 1, b_last)
        for b in range(_NBUF):
            wait_out(b)

    flat_idx = category_ids.reshape(num_idx)
    return _gather(weight, flat_idx)
